# baseline (speedup 1.0000x reference)
import jax
import jax.numpy as jnp
from jax.experimental import pallas as pl
from jax.experimental.pallas import tpu as pltpu

_C = 16
_N = 1048576
_BLK = 8192


def _body(loss_ref, succ_ref, out_ref, acc_ref):
    i = pl.program_id(0)

    @pl.when(i == 0)
    def _init():
        acc_ref[...] = jnp.zeros_like(acc_ref)

    x = loss_ref[...]
    masked = jnp.where(succ_ref[...] == 1, x, 0.0)
    true_s = jnp.sum(masked, axis=1, keepdims=True)
    tot_s = jnp.sum(x, axis=1, keepdims=True)
    acc_ref[...] += jnp.concatenate([true_s, tot_s], axis=1)

    @pl.when(i == pl.num_programs(0) - 1)
    def _fini():
        ts = acc_ref[:, 0:1]
        tt = acc_ref[:, 1:2]
        lv = jnp.log(ts / tt)
        kl = jnp.maximum(lv * lv - 0.01, 0.0)
        out_ref[...] = jnp.sum(kl, axis=0, keepdims=True)


def kernel(lossTensor, lcSuccesses):
    grid = _N // _BLK
    out = pl.pallas_call(
        _body,
        grid=(grid,),
        in_specs=[
            pl.BlockSpec((_C, _BLK), lambda i: (0, i)),
            pl.BlockSpec((_C, _BLK), lambda i: (0, i)),
        ],
        out_specs=pl.BlockSpec((1, 1), lambda i: (0, 0)),
        out_shape=jax.ShapeDtypeStruct((1, 1), jnp.float32),
        scratch_shapes=[pltpu.VMEM((_C, 2), jnp.float32)],
        compiler_params=pltpu.CompilerParams(
            dimension_semantics=("arbitrary",),
        ),
    )(lossTensor, lcSuccesses)
    return out[0, 0]
